# baseline (speedup 1.0000x reference)
import jax
import jax.numpy as jnp
from jax import lax
from jax.experimental import pallas as pl
from jax.experimental.pallas import tpu as pltpu

ROWS = 128
COLS = 32768
BLK_COLS = 4096
GRID = COLS // BLK_COLS
BIG = COLS


def _tc_body(x_ref, out_ref, accm, acci):
    j = pl.program_id(0)
    xb = x_ref[...]
    lmax = jnp.max(xb, axis=1, keepdims=True)
    col = lax.broadcasted_iota(jnp.int32, xb.shape, 1) + j * BLK_COLS
    cand = jnp.where(xb == lmax, col, jnp.int32(BIG))
    larg = jnp.min(cand, axis=1, keepdims=True)

    @pl.when(j == 0)
    def _():
        accm[...] = lmax
        acci[...] = larg

    @pl.when(j > 0)
    def _():
        pm = accm[...]
        pi_ = acci[...]
        upd = lmax > pm
        accm[...] = jnp.where(upd, lmax, pm)
        acci[...] = jnp.where(upd, larg, pi_)

    @pl.when(j == GRID - 1)
    def _():
        out_ref[...] = acci[...].reshape(ROWS)


@jax.jit
def _tc_argmax(x):
    return pl.pallas_call(
        _tc_body,
        grid=(GRID,),
        in_specs=[pl.BlockSpec((ROWS, BLK_COLS), lambda j: (0, j))],
        out_specs=pl.BlockSpec((ROWS,), lambda j: (0,)),
        out_shape=jax.ShapeDtypeStruct((ROWS,), jnp.int32),
        scratch_shapes=[
            pltpu.VMEM((ROWS, 1), jnp.float32),
            pltpu.VMEM((ROWS, 1), jnp.int32),
        ],
    )(x)


def kernel(x):
    return _tc_argmax(x).astype(jnp.int64)


# speedup vs baseline: 1.2476x; 1.2476x over previous
import jax
import jax.numpy as jnp
from jax import lax
from jax.experimental import pallas as pl
from jax.experimental.pallas import tpu as pltpu

ROWS = 128
COLS = 32768
BLK_ROWS = 64
GRID = ROWS // BLK_ROWS


def _tc_body(x_ref, out_ref, acc):
    i = pl.program_id(0)
    xb = x_ref[...]
    rowmax = jnp.max(xb, axis=1, keepdims=True)
    col = lax.broadcasted_iota(jnp.int32, xb.shape, 1)
    cand = jnp.where(xb == rowmax, col, jnp.int32(COLS))
    acc[pl.ds(i * BLK_ROWS, BLK_ROWS), :] = jnp.min(cand, axis=1, keepdims=True)

    @pl.when(i == GRID - 1)
    def _():
        out_ref[...] = acc[...].reshape(ROWS)


@jax.jit
def _tc_argmax(x):
    return pl.pallas_call(
        _tc_body,
        grid=(GRID,),
        in_specs=[pl.BlockSpec((BLK_ROWS, COLS), lambda i: (i, 0))],
        out_specs=pl.BlockSpec((ROWS,), lambda i: (0,)),
        out_shape=jax.ShapeDtypeStruct((ROWS,), jnp.int32),
        scratch_shapes=[pltpu.VMEM((ROWS, 1), jnp.int32)],
    )(x)


def kernel(x):
    return _tc_argmax(x).astype(jnp.int64)
